# baseline (speedup 1.0000x reference)
import functools

import jax
import jax.numpy as jnp
from jax import lax
from jax.experimental import pallas as pl
from jax.experimental.pallas import tpu as pltpu

B, C, H, W = 2, 128, 360, 360
NC = 10
NCP = 16
P = 200
BH = 8
HP, WP = H + 2, W + 2


def _conv_tile(xm_ref, xt_ref, w1_ref, b1_ref, w2_ref, b2_ref, heat_ref):
    x = jnp.concatenate([xm_ref[0], xt_ref[0]], axis=0)
    cols = [x[dy:dy + BH, dx:dx + W, :].reshape(BH * W, C)
            for dy in range(3) for dx in range(3)]
    im2col = jnp.concatenate(cols, axis=1)
    wcat = w1_ref[...].reshape(9 * C, C)
    acc = jnp.dot(im2col, wcat, preferred_element_type=jnp.float32)
    h = jnp.maximum(acc + b1_ref[0], 0.0)
    logits = jnp.dot(h, w2_ref[...],
                     preferred_element_type=jnp.float32) + b2_ref[0]
    heat = jax.nn.sigmoid(logits)
    heat_ref[0] = heat.reshape(BH, W, NCP)


def _heatmap(xp, w1r, b1, w2p, b2p):
    grid = (B, H // BH)
    return pl.pallas_call(
        _conv_tile,
        grid=grid,
        in_specs=[
            pl.BlockSpec((1, BH, WP, C), lambda b, i: (b, i, 0, 0)),
            pl.BlockSpec((1, 2, WP, C), lambda b, i: (b, (i + 1) * BH // 2, 0, 0)),
            pl.BlockSpec((3, 3, C, C), lambda b, i: (0, 0, 0, 0)),
            pl.BlockSpec((1, C), lambda b, i: (0, 0)),
            pl.BlockSpec((C, NCP), lambda b, i: (0, 0)),
            pl.BlockSpec((1, NCP), lambda b, i: (0, 0)),
        ],
        out_specs=pl.BlockSpec((1, BH, W, NCP), lambda b, i: (b, i, 0, 0)),
        out_shape=jax.ShapeDtypeStruct((B, H, W, NCP), jnp.float32),
    )(xp, xp, w1r, b1, w2p, b2p)


def kernel(x, W1, b1, W2, b2, classW, classb, HW1, Hb1, HW2, Hb2):
    xt = x.transpose(0, 2, 3, 1)
    xp = jnp.pad(xt, ((0, 0), (1, 1), (1, 1), (0, 0)))
    w1r = W1.transpose(2, 3, 1, 0)
    w2 = W2[:, :, 0, 0].T
    w2p = jnp.pad(w2, ((0, 0), (0, NCP - NC)))
    b2p = jnp.pad(b2, (0, NCP - NC), constant_values=-1e30)

    heat16 = _heatmap(xp, w1r, b1[None], w2p, b2p[None])
    heatmap = heat16[..., :NC].transpose(0, 3, 1, 2)

    padding = 3 // 2
    inner = lax.reduce_window(heatmap, -jnp.inf, lax.max,
                              (1, 1, 3, 3), (1, 1, 1, 1), 'VALID')
    local_max = jnp.zeros_like(heatmap)
    local_max = local_max.at[:, :, padding:-padding, padding:-padding].set(inner)
    local_max = local_max.at[:, 8].set(heatmap[:, 8])
    local_max = local_max.at[:, 9].set(heatmap[:, 9])
    heatmap_nms = heatmap * (heatmap == local_max).astype(heatmap.dtype)
    flat = heatmap_nms.reshape(B, -1)
    top_scores, top_idx = lax.top_k(flat, P)
    hw = H * W
    top_cls = top_idx // hw
    top_pos = top_idx % hw
    x_flat = x.reshape(B, C, hw)
    idx_b = jnp.broadcast_to(top_pos[:, None, :], (B, C, P))
    query_feat = jnp.take_along_axis(x_flat, idx_b, axis=2)
    one_hot = jax.nn.one_hot(top_cls, NC, dtype=x.dtype).transpose(0, 2, 1)
    cat_enc = jnp.einsum('oc,bcp->bop', classW, one_hot) + classb[None, :, None]
    query_feat = query_feat + cat_enc
    hidden = jax.nn.relu(jnp.einsum('ghc,bcp->bghp', HW1, query_feat)
                         + Hb1[None, :, :, None])
    out = jnp.einsum('goh,bghp->bgop', HW2, hidden) + Hb2[None, :, :, None]
    center = out[:, 0, :2]
    height = out[:, 1, :1]
    dim = out[:, 2, :3]
    rot = out[:, 3, :2]
    vel = out[:, 4, :2]
    hm = out[:, 5, :10]
    return jnp.concatenate([center, height, dim, rot, vel, hm], axis=1)
